# baseline (speedup 1.0000x reference)
import functools

import jax
import jax.numpy as jnp
from jax import lax
from jax.experimental import pallas as pl
from jax.experimental.pallas import tpu as pltpu
from jax.experimental.pallas import tpu_sc as plsc

N = 10000
E = 320000
F1 = 128
F2 = 64
NOUT = 16

NC = 2
NS = 16
NW = NC * NS
CH = 128
EP = 327680
RPW = EP // (NW * CH)
CHA = 64
RPWA = EP // (NW * CHA)
IBRA = 32
DEPTH = 4
IBR = 16
EC = EP // CH
NP = 10240
STR = NP // NS
PAD_DST = 10016

BN = 1000
NB = N // BN
FCB = BN * F2


def _sc_mesh():
    return plsc.VectorSubcoreMesh(
        core_axis_name="c", subcore_axis_name="s",
        num_cores=NC, num_subcores=NS)


def _deg_partials(dst2, znp, ones):
    @functools.partial(
        pl.kernel,
        out_type=jax.ShapeDtypeStruct((NC, NP), jnp.float32),
        mesh=_sc_mesh(),
        scratch_types=[
            pltpu.VMEM((RPW, CH), jnp.int32),
            pltpu.VMEM((CH,), jnp.float32),
            pltpu.VMEM_SHARED((NP,), jnp.float32),
        ],
    )
    def k(dst_hbm, z_hbm, ones_hbm, out_hbm, idx_v, ones_v, acc_sh):
        cid = lax.axis_index("c")
        sid = lax.axis_index("s")
        wid = sid * NC + cid
        pltpu.sync_copy(dst_hbm.at[pl.ds(wid * RPW, RPW)], idx_v)
        pltpu.sync_copy(ones_hbm, ones_v)
        pltpu.sync_copy(z_hbm.at[pl.ds(sid * STR, STR)],
                        acc_sh.at[pl.ds(sid * STR, STR)])
        plsc.subcore_barrier()

        def body(j, carry):
            pltpu.sync_copy(ones_v, acc_sh.at[idx_v.at[j]], add=True)
            return carry

        lax.fori_loop(0, RPW, body, 0)
        plsc.subcore_barrier()
        pltpu.sync_copy(acc_sh.at[pl.ds(sid * STR, STR)],
                        out_hbm.at[cid, pl.ds(sid * STR, STR)])

    return k(dst2, znp, ones)


def _agg_partials(src2, dst2, y, znpf, f):
    @functools.partial(
        pl.kernel,
        out_type=jax.ShapeDtypeStruct((NC, NP, f), jnp.float32),
        mesh=_sc_mesh(),
        scratch_types=[
            pltpu.VMEM((IBRA, CHA), jnp.int32),
            pltpu.VMEM((IBRA, CHA), jnp.int32),
            pltpu.VMEM((DEPTH, CHA, f), jnp.float32),
            pltpu.VMEM_SHARED((NP, f), jnp.float32),
        ] + [pltpu.SemaphoreType.DMA] * DEPTH,
    )
    def k(src_hbm, dst_hbm, y_hbm, z_hbm, out_hbm,
          src_v, dst_v, rows, acc_sh, *sems):
        cid = lax.axis_index("c")
        sid = lax.axis_index("s")
        wid = sid * NC + cid
        pltpu.sync_copy(z_hbm.at[pl.ds(sid * STR, STR)],
                        acc_sh.at[pl.ds(sid * STR, STR)])
        plsc.subcore_barrier()

        def step(j, b):
            pltpu.make_async_copy(
                y_hbm.at[src_v.at[j]], rows.at[b], sems[b]).wait()
            pltpu.sync_copy(rows.at[b], acc_sh.at[dst_v.at[j]], add=True)

            @pl.when(j + DEPTH < IBRA)
            def _():
                pltpu.async_copy(
                    y_hbm.at[src_v.at[j + DEPTH]], rows.at[b], sems[b])

        def inner(i, carry):
            for b in range(DEPTH):
                step(DEPTH * i + b, b)
            return carry

        def block(ib, carry):
            base = wid * RPWA + ib * IBRA
            pltpu.sync_copy(src_hbm.at[pl.ds(base, IBRA)], src_v)
            pltpu.sync_copy(dst_hbm.at[pl.ds(base, IBRA)], dst_v)
            for b in range(DEPTH):
                pltpu.async_copy(y_hbm.at[src_v.at[b]], rows.at[b], sems[b])
            lax.fori_loop(0, IBRA // DEPTH, inner, 0)
            return carry

        lax.fori_loop(0, RPWA // IBRA, block, 0)
        plsc.subcore_barrier()
        pltpu.sync_copy(acc_sh.at[pl.ds(sid * STR, STR)],
                        out_hbm.at[cid, pl.ds(sid * STR, STR)])

    return k(src2, dst2, y, znpf)


def _edge_prep(ei3, per_w, pw_pad):
    def body(es_ref, ed_ref, s_ref, da_ref, dd_ref):
        w = pl.program_id(0)
        i = lax.broadcasted_iota(jnp.int32, (1, 1, pw_pad), 2)
        k = w * pw_pad + i
        s_ref[...] = jnp.concatenate(
            [es_ref[...], N + (k % (NP - N))], axis=2)
        da_ref[...] = jnp.concatenate(
            [ed_ref[...], (k * 1283) % N], axis=2)
        dd_ref[...] = jnp.concatenate(
            [ed_ref[...], PAD_DST + (k % 128)], axis=2)

    full = per_w + pw_pad
    return pl.pallas_call(
        body,
        grid=(NW,),
        in_specs=[
            pl.BlockSpec((1, 1, per_w), lambda w: (w, 0, 0)),
            pl.BlockSpec((1, 1, per_w), lambda w: (NW + w, 0, 0)),
        ],
        out_specs=[
            pl.BlockSpec((1, 1, full), lambda w: (w, 0, 0)),
            pl.BlockSpec((1, 1, full), lambda w: (w, 0, 0)),
            pl.BlockSpec((1, 1, full), lambda w: (w, 0, 0)),
        ],
        out_shape=[
            jax.ShapeDtypeStruct((NW, 1, full), jnp.int32),
            jax.ShapeDtypeStruct((NW, 1, full), jnp.int32),
            jax.ShapeDtypeStruct((NW, 1, full), jnp.int32),
        ],
    )(ei3, ei3)


def _layer1_dense(x, W1, degp3, b1r):
    def body(x_ref, w_ref, dp_ref, b_ref, y_ref, st_ref, dis_ref):
        xw = jnp.dot(x_ref[...], w_ref[...],
                     preferred_element_type=jnp.float32,
                     precision=lax.Precision.HIGHEST)
        deg = dp_ref[0] + dp_ref[1] + 1.0
        dis = lax.rsqrt(deg)
        y_ref[...] = xw * dis
        st_ref[...] = xw * (dis * dis) + b_ref[...]
        dis_ref[...] = dis

    return pl.pallas_call(
        body,
        grid=(NB,),
        in_specs=[
            pl.BlockSpec((BN, F1), lambda j: (j, 0)),
            pl.BlockSpec((F1, F1), lambda j: (0, 0)),
            pl.BlockSpec((NC, BN, 1), lambda j: (0, j, 0)),
            pl.BlockSpec((1, F1), lambda j: (0, 0)),
        ],
        out_specs=[
            pl.BlockSpec((BN, F1), lambda j: (j, 0)),
            pl.BlockSpec((BN, F1), lambda j: (j, 0)),
            pl.BlockSpec((BN, 1), lambda j: (j, 0)),
        ],
        out_shape=[
            jax.ShapeDtypeStruct((N, F1), jnp.float32),
            jax.ShapeDtypeStruct((N, F1), jnp.float32),
            jax.ShapeDtypeStruct((N, 1), jnp.float32),
        ],
    )(x, W1, degp3, b1r)


def _layer2_dense(aggp1, st1, dis, W2, b2r):
    def body(ap_ref, st_ref, dis_ref, w_ref, b_ref, y_ref, s2_ref):
        dis_b = dis_ref[...]
        h1 = jnp.tanh(dis_b * (ap_ref[0] + ap_ref[1]) + st_ref[...])
        xw = jnp.dot(h1, w_ref[...],
                     preferred_element_type=jnp.float32,
                     precision=lax.Precision.HIGHEST)
        y_ref[...] = jnp.concatenate(
            [xw * dis_b, jnp.zeros((BN, F1 - F2), jnp.float32)], axis=1)
        s2_ref[...] = xw * (dis_b * dis_b) + b_ref[...]

    return pl.pallas_call(
        body,
        grid=(NB,),
        in_specs=[
            pl.BlockSpec((NC, BN, F1), lambda j: (0, j, 0)),
            pl.BlockSpec((BN, F1), lambda j: (j, 0)),
            pl.BlockSpec((BN, 1), lambda j: (j, 0)),
            pl.BlockSpec((F1, F2), lambda j: (0, 0)),
            pl.BlockSpec((1, F2), lambda j: (0, 0)),
        ],
        out_specs=[
            pl.BlockSpec((BN, F1), lambda j: (j, 0)),
            pl.BlockSpec((BN, F2), lambda j: (j, 0)),
        ],
        out_shape=[
            jax.ShapeDtypeStruct((N, F1), jnp.float32),
            jax.ShapeDtypeStruct((N, F2), jnp.float32),
        ],
    )(aggp1, st1, dis, W2, b2r)


def _layer2_post(aggp2, st2, dis):
    def body(ap_ref, st_ref, dis_ref, h_ref):
        dis_b = dis_ref[...]
        agg = (ap_ref[0] + ap_ref[1])[:, :F2]
        h_ref[...] = jnp.tanh(dis_b * agg + st_ref[...])

    return pl.pallas_call(
        body,
        grid=(NB,),
        in_specs=[
            pl.BlockSpec((NC, BN, F1), lambda j: (0, j, 0)),
            pl.BlockSpec((BN, F2), lambda j: (j, 0)),
            pl.BlockSpec((BN, 1), lambda j: (j, 0)),
        ],
        out_specs=pl.BlockSpec((BN, F2), lambda j: (j, 0)),
        out_shape=jax.ShapeDtypeStruct((N, F2), jnp.float32),
    )(aggp2, st2, dis)


def _final_fc(h2f, Wfc, bfcr):
    def body(hf_ref, w_ref, b_ref, o_ref):
        j = pl.program_id(0)

        @pl.when(j == 0)
        def _():
            o_ref[...] = b_ref[...]

        o_ref[...] += jnp.sum(hf_ref[...] * w_ref[...], axis=1, keepdims=True)

    return pl.pallas_call(
        body,
        grid=(NB,),
        in_specs=[
            pl.BlockSpec((1, FCB), lambda j: (0, j)),
            pl.BlockSpec((NOUT, FCB), lambda j: (0, j)),
            pl.BlockSpec((NOUT, 1), lambda j: (0, 0)),
        ],
        out_specs=pl.BlockSpec((NOUT, 1), lambda j: (0, 0)),
        out_shape=jax.ShapeDtypeStruct((NOUT, 1), jnp.float32),
    )(h2f, Wfc, bfcr)


def kernel(x, edge_index, batch, W1, b1, W2, b2, Wfc, bfc):
    del batch
    f32 = jnp.float32

    per_w = E // NW
    pw_pad = RPW * CH - per_w
    ei3 = edge_index.reshape(2 * NW, 1, per_w)
    s2w, da2w, dd2w = _edge_prep(ei3, per_w, pw_pad)
    src2 = s2w.reshape(EC * 2, CHA)
    dst2a = da2w.reshape(EC * 2, CHA)
    dst2d = dd2w.reshape(EC, CH)

    ones = jnp.ones((CH,), f32)
    znp = jnp.zeros((NP,), f32)
    znp1 = jnp.zeros((NP, F1), f32)
    zrows1 = jnp.zeros((NP - N, F1), f32)

    degp = _deg_partials(dst2d, znp, ones)
    degp3 = degp.reshape(NC, NP, 1)

    y1, st1, dis = _layer1_dense(x, W1, degp3, b1.reshape(1, F1))
    aggp1 = _agg_partials(src2, dst2a, jnp.concatenate([y1, zrows1]),
                          znp1, F1)

    y2, st2 = _layer2_dense(aggp1, st1, dis, W2, b2.reshape(1, F2))
    aggp2 = _agg_partials(src2, dst2a, jnp.concatenate([y2, zrows1]),
                          znp1, F1)

    h2 = _layer2_post(aggp2, st2, dis)

    out = _final_fc(h2.reshape(1, N * F2), Wfc, bfc.reshape(NOUT, 1))
    return out.reshape(1, NOUT)


# speedup vs baseline: 1.0252x; 1.0252x over previous
import functools

import jax
import jax.numpy as jnp
from jax import lax
from jax.experimental import pallas as pl
from jax.experimental.pallas import tpu as pltpu
from jax.experimental.pallas import tpu_sc as plsc

N = 10000
E = 320000
F1 = 128
F2 = 64
NOUT = 16

NC = 2
NS = 16
NW = NC * NS
CH = 128
EP = 327680
RPW = EP // (NW * CH)
CHA = 64
RPWA = EP // (NW * CHA)
IBRA = 32
DEPTH = 4
IBR = 16
EC = EP // CH
NP = 10240
STR = NP // NS
PAD_DST = 10016

BN = 1000
NB = N // BN
FCB = BN * F2


def _sc_mesh():
    return plsc.VectorSubcoreMesh(
        core_axis_name="c", subcore_axis_name="s",
        num_cores=NC, num_subcores=NS)


def _deg_partials(dst2, znp, ones):
    @functools.partial(
        pl.kernel,
        out_type=jax.ShapeDtypeStruct((NC, NP), jnp.float32),
        mesh=_sc_mesh(),
        scratch_types=[
            pltpu.VMEM((RPW, CH), jnp.int32),
            pltpu.VMEM((CH,), jnp.float32),
            pltpu.VMEM_SHARED((NP,), jnp.float32),
        ],
    )
    def k(dst_hbm, z_hbm, ones_hbm, out_hbm, idx_v, ones_v, acc_sh):
        cid = lax.axis_index("c")
        sid = lax.axis_index("s")
        wid = sid * NC + cid
        pltpu.sync_copy(dst_hbm.at[pl.ds(wid * RPW, RPW)], idx_v)
        pltpu.sync_copy(ones_hbm, ones_v)
        pltpu.sync_copy(z_hbm.at[pl.ds(sid * STR, STR)],
                        acc_sh.at[pl.ds(sid * STR, STR)])
        plsc.subcore_barrier()

        def body(j, carry):
            pltpu.sync_copy(ones_v, acc_sh.at[idx_v.at[j]], add=True)
            return carry

        lax.fori_loop(0, RPW, body, 0)
        plsc.subcore_barrier()
        pltpu.sync_copy(acc_sh.at[pl.ds(sid * STR, STR)],
                        out_hbm.at[cid, pl.ds(sid * STR, STR)])

    return k(dst2, znp, ones)


def _agg_partials(src2, dst2, y, znpf, f):
    @functools.partial(
        pl.kernel,
        out_type=jax.ShapeDtypeStruct((NC, NP, f), jnp.float32),
        mesh=_sc_mesh(),
        scratch_types=[
            pltpu.VMEM((IBRA, CHA), jnp.int32),
            pltpu.VMEM((IBRA, CHA), jnp.int32),
            pltpu.VMEM((DEPTH, CHA, f), jnp.float32),
            pltpu.VMEM_SHARED((NP, f), jnp.float32),
        ] + [pltpu.SemaphoreType.DMA] * DEPTH,
    )
    def k(src_hbm, dst_hbm, y_hbm, z_hbm, out_hbm,
          src_v, dst_v, rows, acc_sh, *sems):
        cid = lax.axis_index("c")
        sid = lax.axis_index("s")
        wid = sid * NC + cid
        pltpu.sync_copy(z_hbm.at[pl.ds(sid * STR, STR)],
                        acc_sh.at[pl.ds(sid * STR, STR)])
        plsc.subcore_barrier()

        def step(j, b):
            pltpu.make_async_copy(
                y_hbm.at[src_v.at[j]], rows.at[b], sems[b]).wait()
            pltpu.sync_copy(rows.at[b], acc_sh.at[dst_v.at[j]], add=True)

            @pl.when(j + DEPTH < IBRA)
            def _():
                pltpu.async_copy(
                    y_hbm.at[src_v.at[j + DEPTH]], rows.at[b], sems[b])

        def inner(i, carry):
            for b in range(DEPTH):
                step(DEPTH * i + b, b)
            return carry

        def block(ib, carry):
            base = wid * RPWA + ib * IBRA
            pltpu.sync_copy(src_hbm.at[pl.ds(base, IBRA)], src_v)
            pltpu.sync_copy(dst_hbm.at[pl.ds(base, IBRA)], dst_v)
            for b in range(DEPTH):
                pltpu.async_copy(y_hbm.at[src_v.at[b]], rows.at[b], sems[b])
            lax.fori_loop(0, IBRA // DEPTH, inner, 0)
            return carry

        lax.fori_loop(0, RPWA // IBRA, block, 0)
        plsc.subcore_barrier()
        pltpu.sync_copy(acc_sh.at[pl.ds(sid * STR, STR)],
                        out_hbm.at[cid, pl.ds(sid * STR, STR)])

    return k(src2, dst2, y, znpf)


def _mm1(x, W1):
    def body(x_ref, w_ref, o_ref):
        o_ref[...] = jnp.dot(x_ref[...], w_ref[...],
                             preferred_element_type=jnp.float32,
                             precision=lax.Precision.HIGHEST)

    return pl.pallas_call(
        body,
        grid=(NB,),
        in_specs=[
            pl.BlockSpec((BN, F1), lambda j: (j, 0)),
            pl.BlockSpec((F1, F1), lambda j: (0, 0)),
        ],
        out_specs=pl.BlockSpec((BN, F1), lambda j: (j, 0)),
        out_shape=jax.ShapeDtypeStruct((N, F1), jnp.float32),
    )(x, W1)


def _layer1_fix(xw, degp3, b1r):
    def body(xw_ref, dp_ref, b_ref, y_ref, st_ref, dis_ref):
        xw = xw_ref[...]
        deg = dp_ref[0] + dp_ref[1] + 1.0
        dis = lax.rsqrt(deg)
        y_ref[...] = xw * dis
        st_ref[...] = xw * (dis * dis) + b_ref[...]
        dis_ref[...] = dis

    return pl.pallas_call(
        body,
        grid=(NB,),
        in_specs=[
            pl.BlockSpec((BN, F1), lambda j: (j, 0)),
            pl.BlockSpec((NC, BN, 1), lambda j: (0, j, 0)),
            pl.BlockSpec((1, F1), lambda j: (0, 0)),
        ],
        out_specs=[
            pl.BlockSpec((BN, F1), lambda j: (j, 0)),
            pl.BlockSpec((BN, F1), lambda j: (j, 0)),
            pl.BlockSpec((BN, 1), lambda j: (j, 0)),
        ],
        out_shape=[
            jax.ShapeDtypeStruct((N, F1), jnp.float32),
            jax.ShapeDtypeStruct((N, F1), jnp.float32),
            jax.ShapeDtypeStruct((N, 1), jnp.float32),
        ],
    )(xw, degp3, b1r)


def _layer2_dense(aggp1, st1, dis, W2, b2r):
    def body(ap_ref, st_ref, dis_ref, w_ref, b_ref, y_ref, s2_ref):
        dis_b = dis_ref[...]
        h1 = jnp.tanh(dis_b * (ap_ref[0] + ap_ref[1]) + st_ref[...])
        xw = jnp.dot(h1, w_ref[...],
                     preferred_element_type=jnp.float32,
                     precision=lax.Precision.HIGHEST)
        y_ref[...] = jnp.concatenate(
            [xw * dis_b, jnp.zeros((BN, F1 - F2), jnp.float32)], axis=1)
        s2_ref[...] = xw * (dis_b * dis_b) + b_ref[...]

    return pl.pallas_call(
        body,
        grid=(NB,),
        in_specs=[
            pl.BlockSpec((NC, BN, F1), lambda j: (0, j, 0)),
            pl.BlockSpec((BN, F1), lambda j: (j, 0)),
            pl.BlockSpec((BN, 1), lambda j: (j, 0)),
            pl.BlockSpec((F1, F2), lambda j: (0, 0)),
            pl.BlockSpec((1, F2), lambda j: (0, 0)),
        ],
        out_specs=[
            pl.BlockSpec((BN, F1), lambda j: (j, 0)),
            pl.BlockSpec((BN, F2), lambda j: (j, 0)),
        ],
        out_shape=[
            jax.ShapeDtypeStruct((N, F1), jnp.float32),
            jax.ShapeDtypeStruct((N, F2), jnp.float32),
        ],
    )(aggp1, st1, dis, W2, b2r)


def _layer2_post(aggp2, st2, dis):
    def body(ap_ref, st_ref, dis_ref, h_ref):
        dis_b = dis_ref[...]
        agg = (ap_ref[0] + ap_ref[1])[:, :F2]
        h_ref[...] = jnp.tanh(dis_b * agg + st_ref[...])

    return pl.pallas_call(
        body,
        grid=(NB,),
        in_specs=[
            pl.BlockSpec((NC, BN, F1), lambda j: (0, j, 0)),
            pl.BlockSpec((BN, F2), lambda j: (j, 0)),
            pl.BlockSpec((BN, 1), lambda j: (j, 0)),
        ],
        out_specs=pl.BlockSpec((BN, F2), lambda j: (j, 0)),
        out_shape=jax.ShapeDtypeStruct((N, F2), jnp.float32),
    )(aggp2, st2, dis)


def _final_fc(h2f, Wfc, bfcr):
    def body(hf_ref, w_ref, b_ref, o_ref):
        j = pl.program_id(0)

        @pl.when(j == 0)
        def _():
            o_ref[...] = b_ref[...]

        o_ref[...] += jnp.sum(hf_ref[...] * w_ref[...], axis=1, keepdims=True)

    return pl.pallas_call(
        body,
        grid=(NB,),
        in_specs=[
            pl.BlockSpec((1, FCB), lambda j: (0, j)),
            pl.BlockSpec((NOUT, FCB), lambda j: (0, j)),
            pl.BlockSpec((NOUT, 1), lambda j: (0, 0)),
        ],
        out_specs=pl.BlockSpec((NOUT, 1), lambda j: (0, 0)),
        out_shape=jax.ShapeDtypeStruct((NOUT, 1), jnp.float32),
    )(h2f, Wfc, bfcr)


def kernel(x, edge_index, batch, W1, b1, W2, b2, Wfc, bfc):
    del batch
    f32 = jnp.float32

    per_w = E // NW
    pw_pad = RPW * CH - per_w
    src_r = edge_index[0].reshape(NW, per_w)
    dst_r = edge_index[1].reshape(NW, per_w)
    k = jnp.arange(NW * pw_pad, dtype=jnp.int32).reshape(NW, pw_pad)
    pad_src = N + (k % (NP - N))
    pad_dst_agg = (k * 1283) % N
    pad_dst_deg = PAD_DST + (k % 128)
    src2 = jnp.concatenate([src_r, pad_src], axis=1).reshape(EC * 2, CHA)
    dst2a = jnp.concatenate([dst_r, pad_dst_agg], axis=1).reshape(EC * 2, CHA)
    dst2d = jnp.concatenate([dst_r, pad_dst_deg], axis=1).reshape(EC, CH)

    ones = jnp.ones((CH,), f32)
    znp = jnp.zeros((NP,), f32)
    znp1 = jnp.zeros((NP, F1), f32)
    zrows1 = jnp.zeros((NP - N, F1), f32)

    xw = _mm1(x, W1)
    degp = _deg_partials(dst2d, znp, ones)
    degp3 = degp.reshape(NC, NP, 1)

    y1, st1, dis = _layer1_fix(xw, degp3, b1.reshape(1, F1))
    aggp1 = _agg_partials(src2, dst2a, jnp.concatenate([y1, zrows1]),
                          znp1, F1)

    y2, st2 = _layer2_dense(aggp1, st1, dis, W2, b2.reshape(1, F2))
    aggp2 = _agg_partials(src2, dst2a, jnp.concatenate([y2, zrows1]),
                          znp1, F1)

    h2 = _layer2_post(aggp2, st2, dis)

    out = _final_fc(h2.reshape(1, N * F2), Wfc, bfc.reshape(NOUT, 1))
    return out.reshape(1, NOUT)


# speedup vs baseline: 1.0329x; 1.0075x over previous
import functools

import jax
import jax.numpy as jnp
from jax import lax
from jax.experimental import pallas as pl
from jax.experimental.pallas import tpu as pltpu
from jax.experimental.pallas import tpu_sc as plsc

N = 10000
E = 320000
F1 = 128
F2 = 64
NOUT = 16

NC = 2
NS = 16
NW = NC * NS
CH = 128
EP = 327680
RPW = EP // (NW * CH)
CHA = 64
RPWA = EP // (NW * CHA)
IBRA = 32
DEPTH = 4
EC = EP // CH
NP = 10240
STR = NP // NS
PAD_DST = 10016

BN = 1000
NB = N // BN
FCB = BN * F2


def _sc_mesh():
    return plsc.VectorSubcoreMesh(
        core_axis_name="c", subcore_axis_name="s",
        num_cores=NC, num_subcores=NS)


def _deg_partials(dst2, znp, ones):
    @functools.partial(
        pl.kernel,
        out_type=jax.ShapeDtypeStruct((NC, NP), jnp.float32),
        mesh=_sc_mesh(),
        scratch_types=[
            pltpu.VMEM((RPW, CH), jnp.int32),
            pltpu.VMEM((CH,), jnp.float32),
            pltpu.VMEM_SHARED((NP,), jnp.float32),
        ],
    )
    def k(dst_hbm, z_hbm, ones_hbm, out_hbm, idx_v, ones_v, acc_sh):
        cid = lax.axis_index("c")
        sid = lax.axis_index("s")
        wid = sid * NC + cid
        pltpu.sync_copy(dst_hbm.at[pl.ds(wid * RPW, RPW)], idx_v)
        pltpu.sync_copy(ones_hbm, ones_v)
        pltpu.sync_copy(z_hbm.at[pl.ds(sid * STR, STR)],
                        acc_sh.at[pl.ds(sid * STR, STR)])
        plsc.subcore_barrier()

        def body(j, carry):
            pltpu.sync_copy(ones_v, acc_sh.at[idx_v.at[j]], add=True)
            return carry

        lax.fori_loop(0, RPW, body, 0)
        plsc.subcore_barrier()
        pltpu.sync_copy(acc_sh.at[pl.ds(sid * STR, STR)],
                        out_hbm.at[cid, pl.ds(sid * STR, STR)])

    return k(dst2, znp, ones)


def _agg_partials(src2, dst2, y, znpf, f):
    @functools.partial(
        pl.kernel,
        out_type=jax.ShapeDtypeStruct((NC, NP, f), jnp.float32),
        mesh=_sc_mesh(),
        scratch_types=[
            pltpu.VMEM((IBRA, CHA), jnp.int32),
            pltpu.VMEM((IBRA, CHA), jnp.int32),
            pltpu.VMEM((DEPTH, CHA, f), jnp.float32),
            pltpu.VMEM_SHARED((NP, f), jnp.float32),
        ] + [pltpu.SemaphoreType.DMA] * DEPTH,
    )
    def k(src_hbm, dst_hbm, y_hbm, z_hbm, out_hbm,
          src_v, dst_v, rows, acc_sh, *sems):
        cid = lax.axis_index("c")
        sid = lax.axis_index("s")
        wid = sid * NC + cid
        pltpu.sync_copy(z_hbm.at[pl.ds(sid * STR, STR)],
                        acc_sh.at[pl.ds(sid * STR, STR)])
        plsc.subcore_barrier()

        def step(j, b):
            pltpu.make_async_copy(
                y_hbm.at[src_v.at[j]], rows.at[b], sems[b]).wait()
            pltpu.sync_copy(rows.at[b], acc_sh.at[dst_v.at[j]], add=True)

            @pl.when(j + DEPTH < IBRA)
            def _():
                pltpu.async_copy(
                    y_hbm.at[src_v.at[j + DEPTH]], rows.at[b], sems[b])

        def inner(i, carry):
            for b in range(DEPTH):
                step(DEPTH * i + b, b)
            return carry

        def block(ib, carry):
            base = wid * RPWA + ib * IBRA
            pltpu.sync_copy(src_hbm.at[pl.ds(base, IBRA)], src_v)
            pltpu.sync_copy(dst_hbm.at[pl.ds(base, IBRA)], dst_v)
            for b in range(DEPTH):
                pltpu.async_copy(y_hbm.at[src_v.at[b]], rows.at[b], sems[b])
            lax.fori_loop(0, IBRA // DEPTH, inner, 0)
            return carry

        lax.fori_loop(0, RPWA // IBRA, block, 0)
        plsc.subcore_barrier()
        pltpu.sync_copy(acc_sh.at[pl.ds(sid * STR, STR)],
                        out_hbm.at[cid, pl.ds(sid * STR, STR)])

    return k(src2, dst2, y, znpf)


def _layer1_dense(x, W1, degp3, b1r):
    def body(x_ref, w_ref, dp_ref, b_ref, y_ref, st_ref, dis_ref):
        xw = jnp.dot(x_ref[...], w_ref[...],
                     preferred_element_type=jnp.float32,
                     precision=lax.Precision.HIGHEST)
        deg = dp_ref[0] + dp_ref[1] + 1.0
        dis = lax.rsqrt(deg)
        y_ref[...] = xw * dis
        st_ref[...] = xw * (dis * dis) + b_ref[...]
        dis_ref[...] = dis

    return pl.pallas_call(
        body,
        grid=(NB,),
        in_specs=[
            pl.BlockSpec((BN, F1), lambda j: (j, 0)),
            pl.BlockSpec((F1, F1), lambda j: (0, 0)),
            pl.BlockSpec((NC, BN, 1), lambda j: (0, j, 0)),
            pl.BlockSpec((1, F1), lambda j: (0, 0)),
        ],
        out_specs=[
            pl.BlockSpec((BN, F1), lambda j: (j, 0)),
            pl.BlockSpec((BN, F1), lambda j: (j, 0)),
            pl.BlockSpec((BN, 1), lambda j: (j, 0)),
        ],
        out_shape=[
            jax.ShapeDtypeStruct((N, F1), jnp.float32),
            jax.ShapeDtypeStruct((N, F1), jnp.float32),
            jax.ShapeDtypeStruct((N, 1), jnp.float32),
        ],
    )(x, W1, degp3, b1r)


def _layer2_dense(aggp1, st1, dis, W2, b2r):
    def body(ap_ref, st_ref, dis_ref, w_ref, b_ref, y_ref, s2_ref):
        dis_b = dis_ref[...]
        h1 = jnp.tanh(dis_b * (ap_ref[0] + ap_ref[1]) + st_ref[...])
        xw = jnp.dot(h1, w_ref[...],
                     preferred_element_type=jnp.float32,
                     precision=lax.Precision.HIGHEST)
        y_ref[...] = jnp.concatenate(
            [xw * dis_b, jnp.zeros((BN, F1 - F2), jnp.float32)], axis=1)
        s2_ref[...] = xw * (dis_b * dis_b) + b_ref[...]

    return pl.pallas_call(
        body,
        grid=(NB,),
        in_specs=[
            pl.BlockSpec((NC, BN, F1), lambda j: (0, j, 0)),
            pl.BlockSpec((BN, F1), lambda j: (j, 0)),
            pl.BlockSpec((BN, 1), lambda j: (j, 0)),
            pl.BlockSpec((F1, F2), lambda j: (0, 0)),
            pl.BlockSpec((1, F2), lambda j: (0, 0)),
        ],
        out_specs=[
            pl.BlockSpec((BN, F1), lambda j: (j, 0)),
            pl.BlockSpec((BN, F2), lambda j: (j, 0)),
        ],
        out_shape=[
            jax.ShapeDtypeStruct((N, F1), jnp.float32),
            jax.ShapeDtypeStruct((N, F2), jnp.float32),
        ],
    )(aggp1, st1, dis, W2, b2r)


def _layer2_post(aggp2, st2, dis):
    def body(ap_ref, st_ref, dis_ref, h_ref):
        dis_b = dis_ref[...]
        agg = (ap_ref[0] + ap_ref[1])[:, :F2]
        h_ref[...] = jnp.tanh(dis_b * agg + st_ref[...])

    return pl.pallas_call(
        body,
        grid=(NB,),
        in_specs=[
            pl.BlockSpec((NC, BN, F1), lambda j: (0, j, 0)),
            pl.BlockSpec((BN, F2), lambda j: (j, 0)),
            pl.BlockSpec((BN, 1), lambda j: (j, 0)),
        ],
        out_specs=pl.BlockSpec((BN, F2), lambda j: (j, 0)),
        out_shape=jax.ShapeDtypeStruct((N, F2), jnp.float32),
    )(aggp2, st2, dis)


def _final_fc(h2f, Wfc, bfcr):
    def body(hf_ref, w_ref, b_ref, o_ref):
        j = pl.program_id(0)

        @pl.when(j == 0)
        def _():
            o_ref[...] = b_ref[...]

        o_ref[...] += jnp.sum(hf_ref[...] * w_ref[...], axis=1, keepdims=True)

    return pl.pallas_call(
        body,
        grid=(NB,),
        in_specs=[
            pl.BlockSpec((1, FCB), lambda j: (0, j)),
            pl.BlockSpec((NOUT, FCB), lambda j: (0, j)),
            pl.BlockSpec((NOUT, 1), lambda j: (0, 0)),
        ],
        out_specs=pl.BlockSpec((NOUT, 1), lambda j: (0, 0)),
        out_shape=jax.ShapeDtypeStruct((NOUT, 1), jnp.float32),
    )(h2f, Wfc, bfcr)


def kernel(x, edge_index, batch, W1, b1, W2, b2, Wfc, bfc):
    del batch
    f32 = jnp.float32

    per_w = E // NW
    pw_pad = RPW * CH - per_w
    src_r = edge_index[0].reshape(NW, per_w)
    dst_r = edge_index[1].reshape(NW, per_w)
    k = jnp.arange(NW * pw_pad, dtype=jnp.int32).reshape(NW, pw_pad)
    pad_src = N + (k % (NP - N))
    pad_dst_agg = (k * 1283) % N
    pad_dst_deg = PAD_DST + (k % 128)
    src2 = jnp.concatenate([src_r, pad_src], axis=1).reshape(EC * 2, CHA)
    dst2a = jnp.concatenate([dst_r, pad_dst_agg], axis=1).reshape(EC * 2, CHA)
    dst2d = jnp.concatenate([dst_r, pad_dst_deg], axis=1).reshape(EC, CH)

    ones = jnp.ones((CH,), f32)
    znp = jnp.zeros((NP,), f32)
    znp1 = jnp.zeros((NP, F1), f32)
    zrows1 = jnp.zeros((NP - N, F1), f32)

    degp = _deg_partials(dst2d, znp, ones)
    degp3 = degp.reshape(NC, NP, 1)

    y1, st1, dis = _layer1_dense(x, W1, degp3, b1.reshape(1, F1))
    aggp1 = _agg_partials(src2, dst2a, jnp.concatenate([y1, zrows1]),
                          znp1, F1)

    y2, st2 = _layer2_dense(aggp1, st1, dis, W2, b2.reshape(1, F2))
    aggp2 = _agg_partials(src2, dst2a, jnp.concatenate([y2, zrows1]),
                          znp1, F1)

    h2 = _layer2_post(aggp2, st2, dis)

    out = _final_fc(h2.reshape(1, N * F2), Wfc, bfc.reshape(NOUT, 1))
    return out.reshape(1, NOUT)
